# baseline (speedup 1.0000x reference)
import functools

import numpy as np
import jax
import jax.numpy as jnp
from jax import lax
from jax.experimental import pallas as pl
from jax.experimental.pallas import tpu as pltpu
from jax.experimental.pallas import tpu_sc as plsc

_BATCH = 16
_MAX_LEN = 512
_H = 256
_VOCAB = 100000
_NS = 4
_ND = 8
_EMB = 64

_LENGTHS = (512 - 28 * np.arange(_BATCH)).astype(np.int64)
_BS = np.array([(_LENGTHS > t).sum() for t in range(_MAX_LEN)], dtype=np.int64)
_CUR = np.concatenate([[0], np.cumsum(_BS)]).astype(np.int32)
_TOTAL = int(_BS.sum())

_TPAD = 4864
_NW = 32
_PER_W = _TPAD // _NW
_NIDXW = _PER_W * _NS
_CHUNK = 256
_NCHUNK = _TPAD // _CHUNK

def _build_parity():
    par = np.zeros((_TPAD, 1), dtype=np.float32)
    for t in range(_MAX_LEN):
        c, bs = int(_CUR[t]), int(_BS[t])
        if t % 2 == 0:
            par[c:c + bs] = 1.0
    return par

_PARITY = _build_parity()

_NSLOT = 8448
_NSCHUNK = _NSLOT // _CHUNK
_PPAD = 5120
_WIN = 272

def _build_redist():
    cur_ext = np.concatenate(
        [_CUR[:_MAX_LEN + 1],
         np.full((_NSLOT // 16 - _MAX_LEN,), _CUR[_MAX_LEN])]).astype(np.int64)
    acol = []
    wcols = np.zeros((_NSLOT, 1), dtype=np.int32)
    for c in range(_NSCHUNK):
        a = int(cur_ext[16 * c]) // 8 * 8
        acol.append(a)
        for r in range(_CHUNK):
            t = 16 * c + r // 16
            w = int(cur_ext[t]) + (r % 16) - a
            assert 0 <= w < _WIN
            wcols[_CHUNK * c + r, 0] = w
        assert a + _WIN <= _PPAD
    return acol, wcols

_ACOL, _WCOLS = _build_redist()

_FSUB = [(0, 96), (96, 56)]
assert sum(n for _, n in _FSUB) == _PER_W


def _sc_gather_kernel(idx_hbm, table_hbm, out_hbm, idx_v, rows_v, sem):
    wid = lax.axis_index("s") * 2 + lax.axis_index("c")
    base = wid * _NIDXW
    pltpu.sync_copy(idx_hbm.at[pl.ds(base, _NIDXW)], idx_v)
    copies = []
    for j in range(_NS):
        for off, n in _FSUB:
            o = j * _PER_W + off
            copies.append(pltpu.async_copy(
                table_hbm.at[j].at[idx_v.at[pl.ds(o, n)]],
                rows_v.at[pl.ds(o, n)],
                sem,
            ))
    for cp in copies:
        cp.wait()
    row0 = wid * _PER_W
    for j in range(_NS):
        pltpu.sync_copy(
            rows_v.at[pl.ds(j * _PER_W, _PER_W)],
            out_hbm.at[pl.ds(row0, _PER_W), pl.ds(j * _EMB, _EMB)],
        )


def _sc_gather(idx, table):
    mesh = plsc.VectorSubcoreMesh(core_axis_name="c", subcore_axis_name="s")
    k = functools.partial(
        pl.kernel,
        out_type=jax.ShapeDtypeStruct((_TPAD, _H), jnp.bfloat16),
        mesh=mesh,
        compiler_params=pltpu.CompilerParams(use_tc_tiling_on_sc=False),
        scratch_types=[
            pltpu.VMEM((_NIDXW,), jnp.int32),
            pltpu.VMEM((_NIDXW, _EMB), jnp.bfloat16),
            pltpu.SemaphoreType.DMA,
        ],
    )(_sc_gather_kernel)
    return k(idx, table)


def _tc_body(wcol_ref, emb_ref, dense_ref, par_ref, wdt_ref, wet_ref, blin_ref,
             pee_ref, poe_ref, b0e_ref, b0o_ref, modd_ref, mevn_ref,
             b1odd_ref, b1evn_ref, wo_ref, bout_ref, out_ref, pre0, slots):
    def p1(c, carry):
        rows = pl.ds(pl.multiple_of(c * _CHUNK, _CHUNK), _CHUNK)
        x = (jnp.dot(dense_ref[rows, :], wdt_ref[...],
                     preferred_element_type=jnp.float32)
             + jnp.dot(emb_ref[rows, :].astype(jnp.bfloat16), wet_ref[...],
                       preferred_element_type=jnp.float32)
             + blin_ref[...])
        inp = jnp.maximum(x, 0.0).astype(jnp.bfloat16)
        pe = jnp.dot(inp, pee_ref[...], preferred_element_type=jnp.float32) \
            + b0e_ref[...]
        po = jnp.dot(inp, poe_ref[...], preferred_element_type=jnp.float32) \
            + b0o_ref[...]
        m = par_ref[rows, :]
        pre0[rows, :] = pe * m + po * (1.0 - m)
        return carry
    lax.fori_loop(0, _NCHUNK, p1, 0)
    pre0[pl.ds(_TPAD, _PPAD - _TPAD), :] = jnp.zeros((_PPAD - _TPAD, _H),
                                                     jnp.float32)

    citota = lax.broadcasted_iota(jnp.int32, (_CHUNK, _WIN), 1)
    for c in range(_NSCHUNK):
        w = wcol_ref[pl.ds(c * _CHUNK, _CHUNK), :]
        p = jnp.where(w == citota, 1.0, 0.0)
        win = pre0[pl.ds(_ACOL[c], _WIN), :]
        slots[pl.ds(c * _CHUNK, _CHUNK), :] = jnp.dot(
            p, win, preferred_element_type=jnp.float32)

    h0 = jnp.tanh(slots[pl.ds(0, 16), :])
    zeros16 = jnp.zeros((16, _H), jnp.float32)
    lenv = 512 - 28 * lax.broadcasted_iota(jnp.int32, (16, 1), 0)

    def p2(i, carry):
        h0c, h1c, acc = carry
        mo = modd_ref[...]
        me = mevn_ref[...]

        def odd_step(h0x, h1x, k):
            u = jnp.concatenate([h0x, h1x], axis=1).astype(jnp.bfloat16)
            g = jnp.dot(u, mo, preferred_element_type=jnp.float32)
            h1y = jnp.tanh(g[:, _H:] + b1odd_ref[...])
            s = pl.ds(pl.multiple_of(16 * k, 16), 16)
            h0y = jnp.tanh(g[:, :_H] + slots[s, :])
            return h0y, h1y

        def even_step(h0x, h1x, k):
            u = jnp.concatenate([h0x, h1x], axis=1).astype(jnp.bfloat16)
            g = jnp.dot(u, me, preferred_element_type=jnp.float32)
            h1y = jnp.tanh(g[:, _H:] + b1evn_ref[...])
            s = pl.ds(pl.multiple_of(16 * k, 16), 16)
            h0y = jnp.tanh(g[:, :_H] + slots[s, :])
            return h0y, h1y

        k0 = 8 * i
        h0x, h1x = h0c, h1c
        for d in range(0, 8, 2):
            h0x, h1x = odd_step(h0x, h1x, k0 + d + 1)
            h0x, h1x = even_step(h0x, h1x, k0 + d + 2)
            acc = jnp.where(lenv == (k0 + d + 2), h1x, acc)
        return (h0x, h1x, acc)

    _, _, acc = lax.fori_loop(0, _MAX_LEN // 8, p2, (h0, zeros16, zeros16))

    s = jnp.sum(acc * wo_ref[...], axis=1, keepdims=True) + bout_ref[...]
    out_ref[...] = 1.0 / (1.0 + jnp.exp(-s))


def _tc_call(wcol, emb_pad, dense_pad, par, WdT, WeT, blin, Pee, Poe, b0e, b0o,
             Modd, Mevn, b1odd, b1evn, wo, bout):
    return pl.pallas_call(
        _tc_body,
        out_shape=jax.ShapeDtypeStruct((16, 1), jnp.float32),
        in_specs=[pl.BlockSpec(memory_space=pltpu.VMEM)] * 17,
        out_specs=pl.BlockSpec(memory_space=pltpu.VMEM),
        scratch_shapes=[
            pltpu.VMEM((_PPAD, _H), jnp.float32),
            pltpu.VMEM((_NSLOT, _H), jnp.float32),
        ],
    )(wcol, emb_pad, dense_pad, par, WdT, WeT, blin, Pee, Poe, b0e, b0o,
      Modd, Mevn, b1odd, b1evn, wo, bout)


def kernel(dense_data, sparse_data, emb_tables, W_lin, b_lin, Wih_ee, Whh_ee,
           bih_ee, bhh_ee, Wih_oe, Whh_oe, bih_oe, bhh_oe, W_out, b_out):
    f32 = jnp.float32

    sp = jnp.pad(sparse_data.astype(jnp.int32), ((0, _TPAD - _TOTAL), (0, 0)))
    idx = sp.reshape(_NW, _PER_W, _NS).transpose(0, 2, 1).reshape(_NW * _NIDXW)

    dense_pad = jnp.pad(dense_data.astype(f32), ((0, _TPAD - _TOTAL), (0, 0)))

    emb_pad = _sc_gather(idx, emb_tables.astype(jnp.bfloat16))

    WdT = W_lin[:, :_ND].T.astype(f32)
    WeT = W_lin[:, _ND:].T.astype(jnp.bfloat16)
    blin = b_lin[None, :].astype(f32)
    Pee = Wih_ee[0].T.astype(jnp.bfloat16)
    Poe = Wih_oe[0].T.astype(jnp.bfloat16)
    b0e = (bih_ee[0] + bhh_ee[0])[None, :].astype(f32)
    b0o = (bih_oe[0] + bhh_oe[0])[None, :].astype(f32)
    z = jnp.zeros((_H, _H), f32)
    Modd = jnp.concatenate([
        jnp.concatenate([Whh_oe[0].T, Wih_ee[1].T], axis=1),
        jnp.concatenate([z, Whh_ee[1].T], axis=1)], axis=0).astype(jnp.bfloat16)
    Mevn = jnp.concatenate([
        jnp.concatenate([Whh_ee[0].T, Wih_oe[1].T], axis=1),
        jnp.concatenate([z, Whh_oe[1].T], axis=1)], axis=0).astype(jnp.bfloat16)
    b1odd = (bih_ee[1] + bhh_ee[1])[None, :].astype(f32)
    b1evn = (bih_oe[1] + bhh_oe[1])[None, :].astype(f32)
    wo = W_out.astype(f32)
    bout = b_out[None, :].astype(f32)
    par = jnp.asarray(_PARITY)
    wcol = jnp.asarray(_WCOLS)

    out = _tc_call(wcol, emb_pad, dense_pad, par, WdT, WeT, blin, Pee, Poe,
                   b0e, b0o, Modd, Mevn, b1odd, b1evn, wo, bout)
    return out[:, 0]


# speedup vs baseline: 1.2419x; 1.2419x over previous
import functools

import numpy as np
import jax
import jax.numpy as jnp
from jax import lax
from jax.experimental import pallas as pl
from jax.experimental.pallas import tpu as pltpu
from jax.experimental.pallas import tpu_sc as plsc

_BATCH = 16
_MAX_LEN = 512
_H = 256
_VOCAB = 100000
_NS = 4
_ND = 8
_EMB = 64

_LENGTHS = (512 - 28 * np.arange(_BATCH)).astype(np.int64)
_BS = np.array([(_LENGTHS > t).sum() for t in range(_MAX_LEN)], dtype=np.int64)
_CUR = np.concatenate([[0], np.cumsum(_BS)]).astype(np.int32)
_TOTAL = int(_BS.sum())

_TPAD = 4864
_NW = 32
_PER_W = _TPAD // _NW
_NIDXW = _PER_W * _NS
_CHUNK = 256
_NCHUNK = _TPAD // _CHUNK

def _build_parity():
    par = np.zeros((_TPAD, 1), dtype=np.float32)
    for t in range(_MAX_LEN):
        c, bs = int(_CUR[t]), int(_BS[t])
        if t % 2 == 0:
            par[c:c + bs] = 1.0
    return par

_PARITY = _build_parity()

_NSLOT = 8448
_NSCHUNK = _NSLOT // _CHUNK
_PPAD = 5120
_WIN = 272

def _build_redist():
    cur_ext = np.concatenate(
        [_CUR[:_MAX_LEN + 1],
         np.full((_NSLOT // 16 - _MAX_LEN,), _CUR[_MAX_LEN])]).astype(np.int64)
    acol = []
    wcols = np.zeros((_NSLOT, 1), dtype=np.int32)
    for c in range(_NSCHUNK):
        a = int(cur_ext[16 * c]) // 8 * 8
        acol.append(a)
        for r in range(_CHUNK):
            t = 16 * c + r // 16
            w = int(cur_ext[t]) + (r % 16) - a
            assert 0 <= w < _WIN
            wcols[_CHUNK * c + r, 0] = w
        assert a + _WIN <= _PPAD
    return acol, wcols

_ACOL, _WCOLS = _build_redist()

_FSUB = [(0, 96), (96, 56)]
assert sum(n for _, n in _FSUB) == _PER_W


def _sc_gather_kernel(idx_hbm, table_hbm, out_hbm, idx_v, rows_v, sem):
    wid = lax.axis_index("s") * 2 + lax.axis_index("c")
    base = wid * _NIDXW
    pltpu.sync_copy(idx_hbm.at[pl.ds(base, _NIDXW)], idx_v)
    copies = []
    for j in range(_NS):
        for off, n in _FSUB:
            o = j * _PER_W + off
            copies.append(pltpu.async_copy(
                table_hbm.at[j].at[idx_v.at[pl.ds(o, n)]],
                rows_v.at[pl.ds(o, n)],
                sem,
            ))
    for cp in copies:
        cp.wait()
    row0 = wid * _PER_W
    for j in range(_NS):
        pltpu.sync_copy(
            rows_v.at[pl.ds(j * _PER_W, _PER_W)],
            out_hbm.at[pl.ds(row0, _PER_W), pl.ds(j * _EMB, _EMB)],
        )


def _sc_gather(idx, table):
    mesh = plsc.VectorSubcoreMesh(core_axis_name="c", subcore_axis_name="s")
    k = functools.partial(
        pl.kernel,
        out_type=jax.ShapeDtypeStruct((_TPAD, _H), jnp.float32),
        mesh=mesh,
        compiler_params=pltpu.CompilerParams(use_tc_tiling_on_sc=False),
        scratch_types=[
            pltpu.VMEM((_NIDXW,), jnp.int32),
            pltpu.VMEM((_NIDXW, _EMB), jnp.float32),
            pltpu.SemaphoreType.DMA,
        ],
    )(_sc_gather_kernel)
    return k(idx, table)


def _tc_body(wcol_ref, emb_ref, dense_ref, par_ref, wdt_ref, wet_ref, blin_ref,
             pee_ref, poe_ref, b0e_ref, b0o_ref, modd_ref, mevn_ref,
             b1odd_ref, b1evn_ref, wo_ref, bout_ref, out_ref, pre0, slots):
    def p1(c, carry):
        rows = pl.ds(pl.multiple_of(c * _CHUNK, _CHUNK), _CHUNK)
        x = (jnp.dot(dense_ref[rows, :], wdt_ref[...],
                     preferred_element_type=jnp.float32)
             + jnp.dot(emb_ref[rows, :].astype(jnp.bfloat16), wet_ref[...],
                       preferred_element_type=jnp.float32)
             + blin_ref[...])
        inp = jnp.maximum(x, 0.0).astype(jnp.bfloat16)
        pe = jnp.dot(inp, pee_ref[...], preferred_element_type=jnp.float32) \
            + b0e_ref[...]
        po = jnp.dot(inp, poe_ref[...], preferred_element_type=jnp.float32) \
            + b0o_ref[...]
        m = par_ref[rows, :]
        pre0[rows, :] = pe * m + po * (1.0 - m)
        return carry
    lax.fori_loop(0, _NCHUNK, p1, 0)
    pre0[pl.ds(_TPAD, _PPAD - _TPAD), :] = jnp.zeros((_PPAD - _TPAD, _H),
                                                     jnp.float32)

    citota = lax.broadcasted_iota(jnp.int32, (_CHUNK, _WIN), 1)
    for c in range(_NSCHUNK):
        w = wcol_ref[pl.ds(c * _CHUNK, _CHUNK), :]
        p = jnp.where(w == citota, 1.0, 0.0)
        win = pre0[pl.ds(_ACOL[c], _WIN), :]
        slots[pl.ds(c * _CHUNK, _CHUNK), :] = jnp.dot(
            p, win, preferred_element_type=jnp.float32)

    h0 = jnp.tanh(slots[pl.ds(0, 16), :])
    zeros16 = jnp.zeros((16, _H), jnp.float32)
    lenv = 512 - 28 * lax.broadcasted_iota(jnp.int32, (16, 1), 0)

    def p2(i, carry):
        h0c, h1c, acc = carry
        mo = modd_ref[...]
        me = mevn_ref[...]

        def odd_step(h0x, h1x, k):
            u = jnp.concatenate([h0x, h1x], axis=1).astype(jnp.bfloat16)
            g = jnp.dot(u, mo, preferred_element_type=jnp.float32)
            h1y = jnp.tanh(g[:, _H:] + b1odd_ref[...])
            s = pl.ds(pl.multiple_of(16 * k, 16), 16)
            h0y = jnp.tanh(g[:, :_H] + slots[s, :])
            return h0y, h1y

        def even_step(h0x, h1x, k):
            u = jnp.concatenate([h0x, h1x], axis=1).astype(jnp.bfloat16)
            g = jnp.dot(u, me, preferred_element_type=jnp.float32)
            h1y = jnp.tanh(g[:, _H:] + b1evn_ref[...])
            s = pl.ds(pl.multiple_of(16 * k, 16), 16)
            h0y = jnp.tanh(g[:, :_H] + slots[s, :])
            return h0y, h1y

        k0 = 8 * i
        h0x, h1x = h0c, h1c
        for d in range(0, 8, 2):
            h0x, h1x = odd_step(h0x, h1x, k0 + d + 1)
            h0x, h1x = even_step(h0x, h1x, k0 + d + 2)
            acc = jnp.where(lenv == (k0 + d + 2), h1x, acc)
        return (h0x, h1x, acc)

    _, _, acc = lax.fori_loop(0, _MAX_LEN // 8, p2, (h0, zeros16, zeros16))

    s = jnp.sum(acc * wo_ref[...], axis=1, keepdims=True) + bout_ref[...]
    out_ref[...] = 1.0 / (1.0 + jnp.exp(-s))


def _tc_call(wcol, emb_pad, dense_pad, par, WdT, WeT, blin, Pee, Poe, b0e, b0o,
             Modd, Mevn, b1odd, b1evn, wo, bout):
    return pl.pallas_call(
        _tc_body,
        out_shape=jax.ShapeDtypeStruct((16, 1), jnp.float32),
        in_specs=[pl.BlockSpec(memory_space=pltpu.VMEM)] * 17,
        out_specs=pl.BlockSpec(memory_space=pltpu.VMEM),
        scratch_shapes=[
            pltpu.VMEM((_PPAD, _H), jnp.float32),
            pltpu.VMEM((_NSLOT, _H), jnp.float32),
        ],
    )(wcol, emb_pad, dense_pad, par, WdT, WeT, blin, Pee, Poe, b0e, b0o,
      Modd, Mevn, b1odd, b1evn, wo, bout)


def kernel(dense_data, sparse_data, emb_tables, W_lin, b_lin, Wih_ee, Whh_ee,
           bih_ee, bhh_ee, Wih_oe, Whh_oe, bih_oe, bhh_oe, W_out, b_out):
    f32 = jnp.float32

    sp = jnp.pad(sparse_data.astype(jnp.int32), ((0, _TPAD - _TOTAL), (0, 0)))
    idx = sp.reshape(_NW, _PER_W, _NS).transpose(0, 2, 1).reshape(_NW * _NIDXW)

    dense_pad = jnp.pad(dense_data.astype(f32), ((0, _TPAD - _TOTAL), (0, 0)))

    emb_pad = _sc_gather(idx, emb_tables.astype(f32))

    WdT = W_lin[:, :_ND].T.astype(f32)
    WeT = W_lin[:, _ND:].T.astype(jnp.bfloat16)
    blin = b_lin[None, :].astype(f32)
    Pee = Wih_ee[0].T.astype(jnp.bfloat16)
    Poe = Wih_oe[0].T.astype(jnp.bfloat16)
    b0e = (bih_ee[0] + bhh_ee[0])[None, :].astype(f32)
    b0o = (bih_oe[0] + bhh_oe[0])[None, :].astype(f32)
    z = jnp.zeros((_H, _H), f32)
    Modd = jnp.concatenate([
        jnp.concatenate([Whh_oe[0].T, Wih_ee[1].T], axis=1),
        jnp.concatenate([z, Whh_ee[1].T], axis=1)], axis=0).astype(jnp.bfloat16)
    Mevn = jnp.concatenate([
        jnp.concatenate([Whh_ee[0].T, Wih_oe[1].T], axis=1),
        jnp.concatenate([z, Whh_oe[1].T], axis=1)], axis=0).astype(jnp.bfloat16)
    b1odd = (bih_ee[1] + bhh_ee[1])[None, :].astype(f32)
    b1evn = (bih_oe[1] + bhh_oe[1])[None, :].astype(f32)
    wo = W_out.astype(f32)
    bout = b_out[None, :].astype(f32)
    par = jnp.asarray(_PARITY)
    wcol = jnp.asarray(_WCOLS)

    out = _tc_call(wcol, emb_pad, dense_pad, par, WdT, WeT, blin, Pee, Poe,
                   b0e, b0o, Modd, Mevn, b1odd, b1evn, wo, bout)
    return out[:, 0]
